# baseline (speedup 1.0000x reference)
import functools

import jax
import jax.numpy as jnp
from jax import lax
from jax.experimental import pallas as pl
from jax.experimental.pallas import tpu as pltpu
from jax.experimental.pallas import tpu_sc as plsc

N0, E0 = 10000, 320000
N1, E1 = 2500, 40000
N2, E2 = 625, 10000
D = 128

NP0, NP1, NP2 = 10240, 2560, 640
EP0, EP1, EP2 = 327680, 40960, 16384
PE0, PE1 = 16384, 8192


def _pad1(a, n, val):
    return jnp.pad(a, (0, n - a.shape[0]), constant_values=val)


def _row_grid(NP):
    R = 2048 if NP % 2048 == 0 else NP
    return R, NP // R


def _rs_of(d_ref):
    deg = d_ref[:, 0:1] + d_ref[:, 1:2]
    return lax.rsqrt(jnp.maximum(deg, 1.0))


def _prep_call(xp, degT):
    NP = xp.shape[0]
    R, g = _row_grid(NP)

    def body(x_ref, d_ref, o_ref):
        o_ref[...] = x_ref[...] * _rs_of(d_ref)

    return pl.pallas_call(
        body,
        grid=(g,),
        in_specs=[
            pl.BlockSpec((R, D), lambda i: (i, 0)),
            pl.BlockSpec((R, 2), lambda i: (i, 0)),
        ],
        out_specs=pl.BlockSpec((R, D), lambda i: (i, 0)),
        out_shape=jax.ShapeDtypeStruct((NP, D), jnp.float32),
    )(xp, degT)


def _dense_call(parts, degT, W, b, relu):
    NP = parts.shape[1]
    R, g = _row_grid(NP)

    def body(p_ref, d_ref, w_ref, b_ref, o_ref):
        agg = p_ref[0] + p_ref[1]
        h = jnp.dot(agg * _rs_of(d_ref), w_ref[...],
                    preferred_element_type=jnp.float32) + b_ref[...]
        o_ref[...] = jnp.maximum(h, 0.0) if relu else h

    return pl.pallas_call(
        body,
        grid=(g,),
        in_specs=[
            pl.BlockSpec((2, R, D), lambda i: (0, i, 0)),
            pl.BlockSpec((R, 2), lambda i: (i, 0)),
            pl.BlockSpec((D, D), lambda i: (0, 0)),
            pl.BlockSpec((1, D), lambda i: (0, 0)),
        ],
        out_specs=pl.BlockSpec((R, D), lambda i: (i, 0)),
        out_shape=jax.ShapeDtypeStruct((NP, D), jnp.float32),
    )(parts, degT, W, b.reshape(1, D))


def _pooldiv_call(parts, cT, degT):
    NP = parts.shape[1]
    R, g = _row_grid(NP)

    def body(p_ref, c_ref, d_ref, o_ref):
        s = p_ref[0] + p_ref[1]
        c = jnp.maximum(c_ref[:, 0:1] + c_ref[:, 1:2], 1.0)
        o_ref[...] = (s / c) * _rs_of(d_ref)

    return pl.pallas_call(
        body,
        grid=(g,),
        in_specs=[
            pl.BlockSpec((2, R, D), lambda i: (0, i, 0)),
            pl.BlockSpec((R, 2), lambda i: (i, 0)),
            pl.BlockSpec((R, 2), lambda i: (i, 0)),
        ],
        out_specs=pl.BlockSpec((R, D), lambda i: (i, 0)),
        out_shape=jax.ShapeDtypeStruct((NP, D), jnp.float32),
    )(parts, cT, degT)


def _unpooladd_call(gparts, skip, degT):
    NP = gparts.shape[1]
    R, g = _row_grid(NP)

    def body(p_ref, s_ref, d_ref, o_ref):
        u = p_ref[0] + p_ref[1] + s_ref[...]
        o_ref[...] = u * _rs_of(d_ref)

    return pl.pallas_call(
        body,
        grid=(g,),
        in_specs=[
            pl.BlockSpec((2, R, D), lambda i: (0, i, 0)),
            pl.BlockSpec((R, D), lambda i: (i, 0)),
            pl.BlockSpec((R, 2), lambda i: (i, 0)),
        ],
        out_specs=pl.BlockSpec((R, D), lambda i: (i, 0)),
        out_shape=jax.ShapeDtypeStruct((NP, D), jnp.float32),
    )(gparts, skip, degT)


_MESH = plsc.VectorSubcoreMesh(core_axis_name="c", subcore_axis_name="s")
NC, NS = 2, 16
K = 128


def _chunk_of(n):
    for c in (128, 80, 64, 40, 32, 20, 16, 8):
        if n % c == 0:
            return c
    raise ValueError(n)


def _zero_fill(ref, nrows):
    z = jnp.zeros((16,), jnp.float32)
    for i in range(nrows):
        for j in range(D // 16):
            ref[i, pl.ds(j * 16, 16)] = z


def _hist_calls(idxs, nbins):
    n_arr = len(idxs)
    out_ty = [jax.ShapeDtypeStruct((2 * nb,), jnp.float32) for nb in nbins]
    scratch = [
        pltpu.VMEM((K,), jnp.float32),
        pltpu.VMEM((K,), jnp.float32),
        pltpu.VMEM((K,), jnp.int32),
        pltpu.VMEM((K,), jnp.float32),
    ] + [pltpu.VMEM_SHARED((nb,), jnp.float32) for nb in nbins]

    @functools.partial(pl.kernel, out_type=out_ty, mesh=_MESH,
                       scratch_types=scratch)
    def k(*refs):
        in_refs = refs[:n_arr]
        out_refs = refs[n_arr:2 * n_arr]
        ones_v, zero_v, idx_v, bounce_v = refs[2 * n_arr:2 * n_arr + 4]
        accs = refs[2 * n_arr + 4:]
        cid = lax.axis_index("c")
        sid = lax.axis_index("s")
        wid = cid * NS + sid
        for j in range(K // 16):
            ones_v[pl.ds(j * 16, 16)] = jnp.ones((16,), jnp.float32)
            zero_v[pl.ds(j * 16, 16)] = jnp.zeros((16,), jnp.float32)
        for a, nb in enumerate(nbins):
            bins_pt = nb // NS
            c = _chunk_of(bins_pt)
            b0 = sid * bins_pt

            def zstep(t, _, a=a, c=c, b0=b0):
                pltpu.sync_copy(zero_v.at[pl.ds(0, c)],
                                accs[a].at[pl.ds(b0 + t * c, c)])
                return 0
            lax.fori_loop(0, bins_pt // c, zstep, 0)
        plsc.subcore_barrier()
        for a, (ep, nb) in enumerate(zip([i.shape[0] for i in idxs], nbins)):
            et = ep // (NC * NS)
            base0 = wid * et

            def estep(g, _, a=a, base0=base0):
                pltpu.sync_copy(in_refs[a].at[pl.ds(base0 + g * K, K)], idx_v)
                pltpu.sync_copy(ones_v, accs[a].at[idx_v], add=True)
                return 0
            lax.fori_loop(0, et // K, estep, 0)
        plsc.subcore_barrier()
        for a, nb in enumerate(nbins):
            bins_pt = nb // NS
            c = _chunk_of(bins_pt)
            b0 = sid * bins_pt

            def dstep(t, _, a=a, c=c, b0=b0, nb=nb):
                pltpu.sync_copy(accs[a].at[pl.ds(b0 + t * c, c)],
                                bounce_v.at[pl.ds(0, c)])
                pltpu.sync_copy(bounce_v.at[pl.ds(0, c)],
                                out_refs[a].at[pl.ds(cid * nb + b0 + t * c, c)])
                return 0
            lax.fori_loop(0, bins_pt // c, dstep, 0)

    return k(*idxs)


def _sc_edge_call(src, dst, table, nacc):
    ep = src.shape[0]
    et = ep // (NC * NS)
    rows_pt = nacc // NS
    c = _chunk_of(rows_pt)

    C = et // K
    ZR = 32 if rows_pt % 32 == 0 else 8
    assert C >= 1, (ep, C)

    @functools.partial(
        pl.kernel,
        out_type=jax.ShapeDtypeStruct((2 * nacc, D), jnp.float32),
        mesh=_MESH,
        scratch_types=[
            pltpu.VMEM((K,), jnp.int32),
            pltpu.VMEM((K,), jnp.int32),
            pltpu.VMEM((K, D), jnp.float32),
            pltpu.VMEM((ZR, D), jnp.float32),
            pltpu.VMEM_SHARED((nacc, D), jnp.float32),
            pltpu.SemaphoreType.DMA,
        ],
    )
    def k(src_hbm, dst_hbm, tab_hbm, out_hbm, sidx, didx, rows, zbuf, acc, sem):
        cid = lax.axis_index("c")
        sid = lax.axis_index("s")
        wid = cid * NS + sid
        _zero_fill(zbuf, ZR)
        r0 = sid * rows_pt

        def zstep(t, _):
            pltpu.sync_copy(zbuf, acc.at[pl.ds(r0 + t * ZR, ZR)])
            return 0
        lax.fori_loop(0, rows_pt // ZR, zstep, 0)
        plsc.subcore_barrier()
        base0 = wid * et

        def estep(g, _):
            b = base0 + g * K
            pltpu.sync_copy(src_hbm.at[pl.ds(b, K)], sidx)
            pltpu.sync_copy(dst_hbm.at[pl.ds(b, K)], didx)
            pltpu.async_copy(tab_hbm.at[sidx], rows, sem).wait()
            pltpu.sync_copy(rows, acc.at[didx], add=True)
            return 0
        lax.fori_loop(0, C, estep, 0)
        plsc.subcore_barrier()

        def dstep(t, _):
            rr = r0 + t * c
            pltpu.sync_copy(acc.at[pl.ds(rr, c)],
                            out_hbm.at[pl.ds(cid * nacc + rr, c)])
            return 0
        lax.fori_loop(0, rows_pt // c, dstep, 0)

    return k(src, dst, table).reshape(2, nacc, D)


def _edge_pass(src, dst, table, nacc):
    return _sc_edge_call(src, dst, table, nacc)


def kernel(x, edge_index_0, edge_index_1, edge_index_2, clusters_0, clusters_1,
           batch, W_d0, b_d0, W_d1, b_d1, W_bot, b_bot, W_u1, b_u1, W_u0, b_u0):
    e0s, e0d = _pad1(edge_index_0[0], EP0, 0), _pad1(edge_index_0[1], EP0, N0)
    e1s, e1d = _pad1(edge_index_1[0], EP1, 0), _pad1(edge_index_1[1], EP1, N1)
    e2s, e2d = _pad1(edge_index_2[0], EP2, 0), _pad1(edge_index_2[1], EP2, N2)
    p0s = _pad1(jnp.arange(N0, dtype=jnp.int32), PE0, 0)
    p0d = _pad1(clusters_0, PE0, N1)
    p1s = _pad1(jnp.arange(N1, dtype=jnp.int32), PE1, 0)
    p1d = _pad1(clusters_1, PE1, N2)
    u1s = _pad1(clusters_1, PE1, 0)
    u1d = _pad1(jnp.arange(N1, dtype=jnp.int32), PE1, N1)
    u0s = _pad1(clusters_0, PE0, 0)
    u0d = _pad1(jnp.arange(N0, dtype=jnp.int32), PE0, N0)
    xp = jnp.pad(x, ((0, NP0 - N0), (0, 0)))

    h0, h1, h2, hc0, hc1 = _hist_calls([e0d, e1d, e2d, p0d, p1d],
                                       [NP0, NP1, NP2, NP1, NP2])
    d0T = h0.reshape(2, NP0).T
    d1T = h1.reshape(2, NP1).T
    d2T = h2.reshape(2, NP2).T
    c0T = hc0.reshape(2, NP1).T
    c1T = hc1.reshape(2, NP2).T

    y0 = _prep_call(xp, d0T)
    a0 = _edge_pass(e0s, e0d, y0, NP0)
    x0 = _dense_call(a0, d0T, W_d0, b_d0, relu=True)
    s1 = _edge_pass(p0s, p0d, x0, NP1)
    y1 = _pooldiv_call(s1, c0T, d1T)
    a1 = _edge_pass(e1s, e1d, y1, NP1)
    x1 = _dense_call(a1, d1T, W_d1, b_d1, relu=True)
    s2 = _edge_pass(p1s, p1d, x1, NP2)
    y2 = _pooldiv_call(s2, c1T, d2T)
    a2 = _edge_pass(e2s, e2d, y2, NP2)
    x2 = _dense_call(a2, d2T, W_bot, b_bot, relu=True)
    g1 = _edge_pass(u1s, u1d, x2, NP1)
    yu1 = _unpooladd_call(g1, x1, d1T)
    au1 = _edge_pass(e1s, e1d, yu1, NP1)
    x1u = _dense_call(au1, d1T, W_u1, b_u1, relu=True)
    g0 = _edge_pass(u0s, u0d, x1u, NP0)
    yu0 = _unpooladd_call(g0, x0, d0T)
    au0 = _edge_pass(e0s, e0d, yu0, NP0)
    out = _dense_call(au0, d0T, W_u0, b_u0, relu=False)
    return out[:N0]


# speedup vs baseline: 1.4105x; 1.4105x over previous
import functools

import jax
import jax.numpy as jnp
from jax import lax
from jax.experimental import pallas as pl
from jax.experimental.pallas import tpu as pltpu
from jax.experimental.pallas import tpu_sc as plsc

N0, E0 = 10000, 320000
N1, E1 = 2500, 40000
N2, E2 = 625, 10000
D = 128

NP0, NP1, NP2 = 10240, 2560, 640
EP0, EP1, EP2 = 327680, 40960, 12288
PE0, PE1 = 12288, 4096


def _pad1(a, n, val):
    return jnp.pad(a, (0, n - a.shape[0]), constant_values=val)


def _row_grid(NP):
    R = 2048 if NP % 2048 == 0 else NP
    return R, NP // R


def _rs_of(d_ref):
    deg = d_ref[:, 0:1] + d_ref[:, 1:2]
    return lax.rsqrt(jnp.maximum(deg, 1.0))


def _prep_call(xp, degT):
    NP = xp.shape[0]
    R, g = _row_grid(NP)

    def body(x_ref, d_ref, o_ref):
        o_ref[...] = x_ref[...] * _rs_of(d_ref)

    return pl.pallas_call(
        body,
        grid=(g,),
        in_specs=[
            pl.BlockSpec((R, D), lambda i: (i, 0)),
            pl.BlockSpec((R, 2), lambda i: (i, 0)),
        ],
        out_specs=pl.BlockSpec((R, D), lambda i: (i, 0)),
        out_shape=jax.ShapeDtypeStruct((NP, D), jnp.float32),
    )(xp, degT)


def _dense_call(parts, degT, W, b, relu):
    NP = parts.shape[1]
    R, g = _row_grid(NP)

    def body(p_ref, d_ref, w_ref, b_ref, o_ref):
        agg = p_ref[0] + p_ref[1]
        h = jnp.dot(agg * _rs_of(d_ref), w_ref[...],
                    preferred_element_type=jnp.float32) + b_ref[...]
        o_ref[...] = jnp.maximum(h, 0.0) if relu else h

    return pl.pallas_call(
        body,
        grid=(g,),
        in_specs=[
            pl.BlockSpec((2, R, D), lambda i: (0, i, 0)),
            pl.BlockSpec((R, 2), lambda i: (i, 0)),
            pl.BlockSpec((D, D), lambda i: (0, 0)),
            pl.BlockSpec((1, D), lambda i: (0, 0)),
        ],
        out_specs=pl.BlockSpec((R, D), lambda i: (i, 0)),
        out_shape=jax.ShapeDtypeStruct((NP, D), jnp.float32),
    )(parts, degT, W, b.reshape(1, D))


def _pooldiv_call(parts, cT, degT):
    NP = parts.shape[1]
    R, g = _row_grid(NP)

    def body(p_ref, c_ref, d_ref, o_ref):
        s = p_ref[0] + p_ref[1]
        c = jnp.maximum(c_ref[:, 0:1] + c_ref[:, 1:2], 1.0)
        o_ref[...] = (s / c) * _rs_of(d_ref)

    return pl.pallas_call(
        body,
        grid=(g,),
        in_specs=[
            pl.BlockSpec((2, R, D), lambda i: (0, i, 0)),
            pl.BlockSpec((R, 2), lambda i: (i, 0)),
            pl.BlockSpec((R, 2), lambda i: (i, 0)),
        ],
        out_specs=pl.BlockSpec((R, D), lambda i: (i, 0)),
        out_shape=jax.ShapeDtypeStruct((NP, D), jnp.float32),
    )(parts, cT, degT)


def _unpooladd_call(gparts, skip, degT):
    NP = gparts.shape[1]
    R, g = _row_grid(NP)

    def body(p_ref, s_ref, d_ref, o_ref):
        u = p_ref[0] + p_ref[1] + s_ref[...]
        o_ref[...] = u * _rs_of(d_ref)

    return pl.pallas_call(
        body,
        grid=(g,),
        in_specs=[
            pl.BlockSpec((2, R, D), lambda i: (0, i, 0)),
            pl.BlockSpec((R, D), lambda i: (i, 0)),
            pl.BlockSpec((R, 2), lambda i: (i, 0)),
        ],
        out_specs=pl.BlockSpec((R, D), lambda i: (i, 0)),
        out_shape=jax.ShapeDtypeStruct((NP, D), jnp.float32),
    )(gparts, skip, degT)


_MESH = plsc.VectorSubcoreMesh(core_axis_name="c", subcore_axis_name="s")
NC, NS = 2, 16
K = 128


def _chunk_of(n):
    for c in (128, 80, 64, 40, 32, 20, 16, 8):
        if n % c == 0:
            return c
    raise ValueError(n)


def _zero_fill(ref, nrows):
    z = jnp.zeros((16,), jnp.float32)
    for i in range(nrows):
        for j in range(D // 16):
            ref[i, pl.ds(j * 16, 16)] = z


def _hist_calls(idxs, nbins):
    n_arr = len(idxs)
    out_ty = [jax.ShapeDtypeStruct((2 * nb,), jnp.float32) for nb in nbins]
    scratch = [
        pltpu.VMEM((K,), jnp.float32),
        pltpu.VMEM((K,), jnp.float32),
        pltpu.VMEM((K,), jnp.int32),
        pltpu.VMEM((K,), jnp.float32),
    ] + [pltpu.VMEM_SHARED((nb,), jnp.float32) for nb in nbins]

    @functools.partial(pl.kernel, out_type=out_ty, mesh=_MESH,
                       scratch_types=scratch)
    def k(*refs):
        in_refs = refs[:n_arr]
        out_refs = refs[n_arr:2 * n_arr]
        ones_v, zero_v, idx_v, bounce_v = refs[2 * n_arr:2 * n_arr + 4]
        accs = refs[2 * n_arr + 4:]
        cid = lax.axis_index("c")
        sid = lax.axis_index("s")
        wid = cid * NS + sid
        for j in range(K // 16):
            ones_v[pl.ds(j * 16, 16)] = jnp.ones((16,), jnp.float32)
            zero_v[pl.ds(j * 16, 16)] = jnp.zeros((16,), jnp.float32)
        for a, nb in enumerate(nbins):
            bins_pt = nb // NS
            c = _chunk_of(bins_pt)
            b0 = sid * bins_pt

            def zstep(t, _, a=a, c=c, b0=b0):
                pltpu.sync_copy(zero_v.at[pl.ds(0, c)],
                                accs[a].at[pl.ds(b0 + t * c, c)])
                return 0
            lax.fori_loop(0, bins_pt // c, zstep, 0)
        plsc.subcore_barrier()
        for a, (ep, nb) in enumerate(zip([i.shape[0] for i in idxs], nbins)):
            et = ep // (NC * NS)
            base0 = wid * et

            def estep(g, _, a=a, base0=base0):
                pltpu.sync_copy(in_refs[a].at[pl.ds(base0 + g * K, K)], idx_v)
                pltpu.sync_copy(ones_v, accs[a].at[idx_v], add=True)
                return 0
            lax.fori_loop(0, et // K, estep, 0)
        plsc.subcore_barrier()
        for a, nb in enumerate(nbins):
            bins_pt = nb // NS
            c = _chunk_of(bins_pt)
            b0 = sid * bins_pt

            def dstep(t, _, a=a, c=c, b0=b0, nb=nb):
                pltpu.sync_copy(accs[a].at[pl.ds(b0 + t * c, c)],
                                bounce_v.at[pl.ds(0, c)])
                pltpu.sync_copy(bounce_v.at[pl.ds(0, c)],
                                out_refs[a].at[pl.ds(cid * nb + b0 + t * c, c)])
                return 0
            lax.fori_loop(0, bins_pt // c, dstep, 0)

    return k(*idxs)


def _sc_edge_call(src, dst, table, nacc):
    ep = src.shape[0]
    et = ep // (NC * NS)
    rows_pt = nacc // NS
    c = _chunk_of(rows_pt)

    C = et // K
    ZR = 8
    assert C >= 1, (ep, C)

    @functools.partial(
        pl.kernel,
        out_type=jax.ShapeDtypeStruct((2 * nacc, D), jnp.float32),
        mesh=_MESH,
        scratch_types=[
            pltpu.VMEM((K,), jnp.int32),
            pltpu.VMEM((K,), jnp.int32),
            pltpu.VMEM((K, D), jnp.float32),
            pltpu.VMEM((ZR, D), jnp.float32),
            pltpu.VMEM_SHARED((nacc, D), jnp.float32),
            pltpu.SemaphoreType.DMA,
        ],
    )
    def k(src_hbm, dst_hbm, tab_hbm, out_hbm, sidx, didx, rows, zbuf, acc, sem):
        cid = lax.axis_index("c")
        sid = lax.axis_index("s")
        wid = cid * NS + sid
        _zero_fill(zbuf, ZR)
        r0 = sid * rows_pt

        def zstep(t, _):
            pltpu.sync_copy(zbuf, acc.at[pl.ds(r0 + t * ZR, ZR)])
            return 0
        lax.fori_loop(0, rows_pt // ZR, zstep, 0)
        plsc.subcore_barrier()
        base0 = wid * et

        def estep(g, _):
            b = base0 + g * K
            pltpu.sync_copy(src_hbm.at[pl.ds(b, K)], sidx)
            pltpu.sync_copy(dst_hbm.at[pl.ds(b, K)], didx)
            pltpu.async_copy(tab_hbm.at[sidx], rows, sem).wait()
            pltpu.sync_copy(rows, acc.at[didx], add=True)
            return 0
        lax.fori_loop(0, C, estep, 0)
        plsc.subcore_barrier()

        def dstep(t, _):
            rr = r0 + t * c
            pltpu.sync_copy(acc.at[pl.ds(rr, c)],
                            out_hbm.at[pl.ds(cid * nacc + rr, c)])
            return 0
        lax.fori_loop(0, rows_pt // c, dstep, 0)

    return k(src, dst, table).reshape(2, nacc, D)


def _edge_pass(src, dst, table, nacc):
    return _sc_edge_call(src, dst, table, nacc)


def kernel(x, edge_index_0, edge_index_1, edge_index_2, clusters_0, clusters_1,
           batch, W_d0, b_d0, W_d1, b_d1, W_bot, b_bot, W_u1, b_u1, W_u0, b_u0):
    e0s, e0d = _pad1(edge_index_0[0], EP0, 0), _pad1(edge_index_0[1], EP0, N0)
    e1s, e1d = _pad1(edge_index_1[0], EP1, 0), _pad1(edge_index_1[1], EP1, N1)
    e2s, e2d = _pad1(edge_index_2[0], EP2, 0), _pad1(edge_index_2[1], EP2, N2)
    p0s = _pad1(jnp.arange(N0, dtype=jnp.int32), PE0, 0)
    p0d = _pad1(clusters_0, PE0, N1)
    p1s = _pad1(jnp.arange(N1, dtype=jnp.int32), PE1, 0)
    p1d = _pad1(clusters_1, PE1, N2)
    u1s = _pad1(clusters_1, PE1, 0)
    u1d = _pad1(jnp.arange(N1, dtype=jnp.int32), PE1, N1)
    u0s = _pad1(clusters_0, PE0, 0)
    u0d = _pad1(jnp.arange(N0, dtype=jnp.int32), PE0, N0)
    xp = jnp.pad(x, ((0, NP0 - N0), (0, 0)))

    h0, h1, h2, hc0, hc1 = _hist_calls([e0d, e1d, e2d, p0d, p1d],
                                       [NP0, NP1, NP2, NP1, NP2])
    d0T = h0.reshape(2, NP0).T
    d1T = h1.reshape(2, NP1).T
    d2T = h2.reshape(2, NP2).T
    c0T = hc0.reshape(2, NP1).T
    c1T = hc1.reshape(2, NP2).T

    y0 = _prep_call(xp, d0T)
    a0 = _edge_pass(e0s, e0d, y0, NP0)
    x0 = _dense_call(a0, d0T, W_d0, b_d0, relu=True)
    s1 = _edge_pass(p0s, p0d, x0, NP1)
    y1 = _pooldiv_call(s1, c0T, d1T)
    a1 = _edge_pass(e1s, e1d, y1, NP1)
    x1 = _dense_call(a1, d1T, W_d1, b_d1, relu=True)
    s2 = _edge_pass(p1s, p1d, x1, NP2)
    y2 = _pooldiv_call(s2, c1T, d2T)
    a2 = _edge_pass(e2s, e2d, y2, NP2)
    x2 = _dense_call(a2, d2T, W_bot, b_bot, relu=True)
    g1 = _edge_pass(u1s, u1d, x2, NP1)
    yu1 = _unpooladd_call(g1, x1, d1T)
    au1 = _edge_pass(e1s, e1d, yu1, NP1)
    x1u = _dense_call(au1, d1T, W_u1, b_u1, relu=True)
    g0 = _edge_pass(u0s, u0d, x1u, NP0)
    yu0 = _unpooladd_call(g0, x0, d0T)
    au0 = _edge_pass(e0s, e0d, yu0, NP0)
    out = _dense_call(au0, d0T, W_u0, b_u0, relu=False)
    return out[:N0]


# speedup vs baseline: 3.4729x; 2.4622x over previous
import functools

import jax
import jax.numpy as jnp
from jax import lax
from jax.experimental import pallas as pl
from jax.experimental.pallas import tpu as pltpu
from jax.experimental.pallas import tpu_sc as plsc

N0, E0 = 10000, 320000
N1, E1 = 2500, 40000
N2, E2 = 625, 10000
D = 128

NP0, NP1, NP2 = 10240, 2560, 640
EP0, EP1, EP2 = 323584, 40960, 12288
PE0, PE1 = 12288, 4096


def _pad1(a, n, val):
    return jnp.pad(a, (0, n - a.shape[0]), constant_values=val)


def _pad_spread(a, n, base, nrows):
    pad = base + (jnp.arange(n - a.shape[0], dtype=jnp.int32) % nrows)
    return jnp.concatenate([a, pad])


def _row_grid(NP):
    R = 2048 if NP % 2048 == 0 else NP
    return R, NP // R


def _rs_of(d_ref):
    deg = d_ref[:, 0:1] + d_ref[:, 1:2]
    return lax.rsqrt(jnp.maximum(deg, 1.0))


def _prep_call(xp, degT):
    NP = xp.shape[0]
    R, g = _row_grid(NP)

    def body(x_ref, d_ref, o_ref):
        o_ref[...] = x_ref[...] * _rs_of(d_ref)

    return pl.pallas_call(
        body,
        grid=(g,),
        in_specs=[
            pl.BlockSpec((R, D), lambda i: (i, 0)),
            pl.BlockSpec((R, 2), lambda i: (i, 0)),
        ],
        out_specs=pl.BlockSpec((R, D), lambda i: (i, 0)),
        out_shape=jax.ShapeDtypeStruct((NP, D), jnp.float32),
    )(xp, degT)


def _dense_call(parts, degT, W, b, relu):
    NP = parts.shape[1]
    R, g = _row_grid(NP)

    def body(p_ref, d_ref, w_ref, b_ref, o_ref):
        agg = p_ref[0] + p_ref[1]
        h = jnp.dot(agg * _rs_of(d_ref), w_ref[...],
                    preferred_element_type=jnp.float32) + b_ref[...]
        o_ref[...] = jnp.maximum(h, 0.0) if relu else h

    return pl.pallas_call(
        body,
        grid=(g,),
        in_specs=[
            pl.BlockSpec((2, R, D), lambda i: (0, i, 0)),
            pl.BlockSpec((R, 2), lambda i: (i, 0)),
            pl.BlockSpec((D, D), lambda i: (0, 0)),
            pl.BlockSpec((1, D), lambda i: (0, 0)),
        ],
        out_specs=pl.BlockSpec((R, D), lambda i: (i, 0)),
        out_shape=jax.ShapeDtypeStruct((NP, D), jnp.float32),
    )(parts, degT, W, b.reshape(1, D))


def _pooldiv_call(parts, cT, degT):
    NP = parts.shape[1]
    R, g = _row_grid(NP)

    def body(p_ref, c_ref, d_ref, o_ref):
        s = p_ref[0] + p_ref[1]
        c = jnp.maximum(c_ref[:, 0:1] + c_ref[:, 1:2], 1.0)
        o_ref[...] = (s / c) * _rs_of(d_ref)

    return pl.pallas_call(
        body,
        grid=(g,),
        in_specs=[
            pl.BlockSpec((2, R, D), lambda i: (0, i, 0)),
            pl.BlockSpec((R, 2), lambda i: (i, 0)),
            pl.BlockSpec((R, 2), lambda i: (i, 0)),
        ],
        out_specs=pl.BlockSpec((R, D), lambda i: (i, 0)),
        out_shape=jax.ShapeDtypeStruct((NP, D), jnp.float32),
    )(parts, cT, degT)


def _unpooladd_call(gparts, skip, degT):
    NP = gparts.shape[1]
    R, g = _row_grid(NP)

    def body(p_ref, s_ref, d_ref, o_ref):
        u = p_ref[0] + p_ref[1] + s_ref[...]
        o_ref[...] = u * _rs_of(d_ref)

    return pl.pallas_call(
        body,
        grid=(g,),
        in_specs=[
            pl.BlockSpec((2, R, D), lambda i: (0, i, 0)),
            pl.BlockSpec((R, D), lambda i: (i, 0)),
            pl.BlockSpec((R, 2), lambda i: (i, 0)),
        ],
        out_specs=pl.BlockSpec((R, D), lambda i: (i, 0)),
        out_shape=jax.ShapeDtypeStruct((NP, D), jnp.float32),
    )(gparts, skip, degT)


_MESH = plsc.VectorSubcoreMesh(core_axis_name="c", subcore_axis_name="s")
NC, NS = 2, 16
K = 128


def _chunk_of(n):
    for c in (128, 80, 64, 40, 32, 20, 16, 8):
        if n % c == 0:
            return c
    raise ValueError(n)


def _zero_fill(ref, nrows):
    z = jnp.zeros((16,), jnp.float32)
    for i in range(nrows):
        for j in range(D // 16):
            ref[i, pl.ds(j * 16, 16)] = z


def _hist_calls(idxs, nbins):
    n_arr = len(idxs)
    out_ty = [jax.ShapeDtypeStruct((2 * nb,), jnp.float32) for nb in nbins]
    scratch = [
        pltpu.VMEM((K,), jnp.float32),
        pltpu.VMEM((K,), jnp.float32),
        pltpu.VMEM((K,), jnp.int32),
        pltpu.VMEM((K,), jnp.float32),
    ] + [pltpu.VMEM_SHARED((nb,), jnp.float32) for nb in nbins]

    @functools.partial(pl.kernel, out_type=out_ty, mesh=_MESH,
                       scratch_types=scratch)
    def k(*refs):
        in_refs = refs[:n_arr]
        out_refs = refs[n_arr:2 * n_arr]
        ones_v, zero_v, idx_v, bounce_v = refs[2 * n_arr:2 * n_arr + 4]
        accs = refs[2 * n_arr + 4:]
        cid = lax.axis_index("c")
        sid = lax.axis_index("s")
        wid = cid * NS + sid
        for j in range(K // 16):
            ones_v[pl.ds(j * 16, 16)] = jnp.ones((16,), jnp.float32)
            zero_v[pl.ds(j * 16, 16)] = jnp.zeros((16,), jnp.float32)
        for a, nb in enumerate(nbins):
            bins_pt = nb // NS
            c = _chunk_of(bins_pt)
            b0 = sid * bins_pt

            def zstep(t, _, a=a, c=c, b0=b0):
                pltpu.sync_copy(zero_v.at[pl.ds(0, c)],
                                accs[a].at[pl.ds(b0 + t * c, c)])
                return 0
            lax.fori_loop(0, bins_pt // c, zstep, 0)
        plsc.subcore_barrier()
        for a, (ep, nb) in enumerate(zip([i.shape[0] for i in idxs], nbins)):
            et = ep // (NC * NS)
            base0 = wid * et

            def estep(g, _, a=a, base0=base0):
                pltpu.sync_copy(in_refs[a].at[pl.ds(base0 + g * K, K)], idx_v)
                pltpu.sync_copy(ones_v, accs[a].at[idx_v], add=True)
                return 0
            lax.fori_loop(0, et // K, estep, 0)
        plsc.subcore_barrier()
        for a, nb in enumerate(nbins):
            bins_pt = nb // NS
            c = _chunk_of(bins_pt)
            b0 = sid * bins_pt

            def dstep(t, _, a=a, c=c, b0=b0, nb=nb):
                pltpu.sync_copy(accs[a].at[pl.ds(b0 + t * c, c)],
                                bounce_v.at[pl.ds(0, c)])
                pltpu.sync_copy(bounce_v.at[pl.ds(0, c)],
                                out_refs[a].at[pl.ds(cid * nb + b0 + t * c, c)])
                return 0
            lax.fori_loop(0, bins_pt // c, dstep, 0)

    return k(*idxs)


def _sc_edge_call(src, dst, table, nacc):
    ep = src.shape[0]
    et = ep // (NC * NS)
    rows_pt = nacc // NS
    c = _chunk_of(rows_pt)

    C = et // K
    ZR = 8
    assert C >= 1, (ep, C)

    @functools.partial(
        pl.kernel,
        out_type=jax.ShapeDtypeStruct((2 * nacc, D), jnp.float32),
        mesh=_MESH,
        scratch_types=[
            pltpu.VMEM((K,), jnp.int32),
            pltpu.VMEM((K,), jnp.int32),
            pltpu.VMEM((K, D), jnp.float32),
            pltpu.VMEM((ZR, D), jnp.float32),
            pltpu.VMEM_SHARED((nacc, D), jnp.float32),
            pltpu.SemaphoreType.DMA,
        ],
    )
    def k(src_hbm, dst_hbm, tab_hbm, out_hbm, sidx, didx, rows, zbuf, acc, sem):
        cid = lax.axis_index("c")
        sid = lax.axis_index("s")
        wid = cid * NS + sid
        _zero_fill(zbuf, ZR)
        r0 = sid * rows_pt

        def zstep(t, _):
            pltpu.sync_copy(zbuf, acc.at[pl.ds(r0 + t * ZR, ZR)])
            return 0
        lax.fori_loop(0, rows_pt // ZR, zstep, 0)
        plsc.subcore_barrier()
        base0 = wid * et

        def estep(g, _):
            b = base0 + g * K
            pltpu.sync_copy(src_hbm.at[pl.ds(b, K)], sidx)
            pltpu.sync_copy(dst_hbm.at[pl.ds(b, K)], didx)
            pltpu.async_copy(tab_hbm.at[sidx], rows, sem).wait()
            pltpu.sync_copy(rows, acc.at[didx], add=True)
            return 0
        lax.fori_loop(0, C, estep, 0)
        plsc.subcore_barrier()

        def dstep(t, _):
            rr = r0 + t * c
            pltpu.sync_copy(acc.at[pl.ds(rr, c)],
                            out_hbm.at[pl.ds(cid * nacc + rr, c)])
            return 0
        lax.fori_loop(0, rows_pt // c, dstep, 0)

    return k(src, dst, table).reshape(2, nacc, D)


def _edge_pass(src, dst, table, nacc):
    return _sc_edge_call(src, dst, table, nacc)


def kernel(x, edge_index_0, edge_index_1, edge_index_2, clusters_0, clusters_1,
           batch, W_d0, b_d0, W_d1, b_d1, W_bot, b_bot, W_u1, b_u1, W_u0, b_u0):
    e0s = _pad_spread(edge_index_0[0], EP0, 0, 256)
    e0d = _pad_spread(edge_index_0[1], EP0, N0, NP0 - N0)
    e1s = _pad_spread(edge_index_1[0], EP1, 0, 256)
    e1d = _pad_spread(edge_index_1[1], EP1, N1, NP1 - N1)
    e2s = _pad_spread(edge_index_2[0], EP2, 0, 256)
    e2d = _pad_spread(edge_index_2[1], EP2, N2, NP2 - N2)
    p0s = _pad_spread(jnp.arange(N0, dtype=jnp.int32), PE0, 0, 256)
    p0d = _pad_spread(clusters_0, PE0, N1, NP1 - N1)
    p1s = _pad_spread(jnp.arange(N1, dtype=jnp.int32), PE1, 0, 256)
    p1d = _pad_spread(clusters_1, PE1, N2, NP2 - N2)
    u1s = _pad_spread(clusters_1, PE1, 0, 256)
    u1d = _pad_spread(jnp.arange(N1, dtype=jnp.int32), PE1, N1, NP1 - N1)
    u0s = _pad_spread(clusters_0, PE0, 0, 256)
    u0d = _pad_spread(jnp.arange(N0, dtype=jnp.int32), PE0, N0, NP0 - N0)
    xp = jnp.pad(x, ((0, NP0 - N0), (0, 0)))

    h0, h1, h2, hc0, hc1 = _hist_calls([e0d, e1d, e2d, p0d, p1d],
                                       [NP0, NP1, NP2, NP1, NP2])
    d0T = h0.reshape(2, NP0).T
    d1T = h1.reshape(2, NP1).T
    d2T = h2.reshape(2, NP2).T
    c0T = hc0.reshape(2, NP1).T
    c1T = hc1.reshape(2, NP2).T

    y0 = _prep_call(xp, d0T)
    a0 = _edge_pass(e0s, e0d, y0, NP0)
    x0 = _dense_call(a0, d0T, W_d0, b_d0, relu=True)
    s1 = _edge_pass(p0s, p0d, x0, NP1)
    y1 = _pooldiv_call(s1, c0T, d1T)
    a1 = _edge_pass(e1s, e1d, y1, NP1)
    x1 = _dense_call(a1, d1T, W_d1, b_d1, relu=True)
    s2 = _edge_pass(p1s, p1d, x1, NP2)
    y2 = _pooldiv_call(s2, c1T, d2T)
    a2 = _edge_pass(e2s, e2d, y2, NP2)
    x2 = _dense_call(a2, d2T, W_bot, b_bot, relu=True)
    g1 = _edge_pass(u1s, u1d, x2, NP1)
    yu1 = _unpooladd_call(g1, x1, d1T)
    au1 = _edge_pass(e1s, e1d, yu1, NP1)
    x1u = _dense_call(au1, d1T, W_u1, b_u1, relu=True)
    g0 = _edge_pass(u0s, u0d, x1u, NP0)
    yu0 = _unpooladd_call(g0, x0, d0T)
    au0 = _edge_pass(e0s, e0d, yu0, NP0)
    out = _dense_call(au0, d0T, W_u0, b_u0, relu=False)
    return out[:N0]
